# baseline (speedup 1.0000x reference)
import jax
import jax.numpy as jnp
from jax.experimental import pallas as pl
from jax.experimental.pallas import tpu as pltpu

_N, _C, _F, _H, _W = 2, 19, 256, 128, 128
_HW = _H * _W
_K = 8192
_BLK = 8192


_ROWS = _BLK // _W


def _mlp_kernel(feat_ref, coarse_ref, wa1, wa2, wa3, wap, out_ref):
    bf = jnp.bfloat16
    nch = 2
    cr = _ROWS // nch
    cb = _BLK // nch
    w1, w2, w3, wp = wa1[...], wa2[...], wa3[...], wap[...]

    def dot(a, b):
        return jax.lax.dot(a, b, preferred_element_type=jnp.float32)

    def prep(s):
        feat = feat_ref[0][:, s * cr:(s + 1) * cr].astype(bf).reshape(_F, cb)
        coarse = coarse_ref[0][:, s * cr:(s + 1) * cr].astype(bf).reshape(_C, cb)
        tail = jnp.concatenate([coarse, jnp.ones((1, cb), bf)], axis=0)
        return jnp.concatenate([feat, tail], axis=0), tail

    xs, tails = zip(*[prep(s) for s in range(nch)])
    xs = list(xs)
    for w in (w1, w2, w3):
        ds = [None] * nch
        for s in range(nch):
            ds[s] = dot(w, xs[s])
            if s >= 1:
                h = jnp.maximum(ds[s - 1], 0.0)
                xs[s - 1] = jnp.concatenate([h.astype(bf), tails[s - 1]], axis=0)
        h = jnp.maximum(ds[nch - 1], 0.0)
        xs[nch - 1] = jnp.concatenate([h.astype(bf), tails[nch - 1]], axis=0)
    for s in range(nch):
        out = dot(wp, xs[s])
        out_ref[0, :, s * cr:(s + 1) * cr] = out.reshape(_C, cr, _W)


def _keys_of(sem_list):
    minint = jnp.int32(-2147483648)
    m1 = sem_list[0]
    m2 = jnp.full((_H, _W), -jnp.inf, jnp.float32)
    for c in range(1, _C):
        v = sem_list[c]
        m2 = jnp.maximum(m2, jnp.minimum(m1, v))
        m1 = jnp.maximum(m1, v)
    unc = m2 - m1
    bits = jax.lax.bitcast_convert_type(unc, jnp.int32)
    return jnp.where(bits >= 0, bits, minint - bits)


def _lane_roll(x, k, axis):
    idx = jax.lax.broadcasted_iota(jnp.int32, (_H, _W), axis)
    return jnp.where(idx >= k, pltpu.roll(x, k, axis=axis), 0)


def _topk_masks(keys_pair):
    minint = jnp.int32(-2147483648)

    def t_body(i, ts):
        bit = jnp.left_shift(jnp.int32(1), jnp.int32(30) - i)
        new = []
        for keys, t in zip(keys_pair, ts):
            cand = t + bit
            cnt = jnp.sum((keys > cand).astype(jnp.int32))
            new.append(jnp.where(cnt >= _K, cand, t))
        return tuple(new)

    ts = jax.lax.fori_loop(0, 31, t_body, (minint, minint), unroll=True)

    masks = []
    for keys, t in zip(keys_pair, ts):
        v_thr = t + jnp.int32(1)
        g = jnp.sum((keys > v_thr).astype(jnp.int32))
        r = _K - g
        ties = (keys == v_thr).astype(jnp.int32)
        cum = ties
        for k in (1, 2, 4, 8, 16, 32, 64):
            cum = cum + _lane_roll(cum, k, 1)
        rows = jax.lax.broadcast_in_dim(cum[:, _W - 1], (_H, _W), (0,))
        rowcum = rows
        for k in (1, 2, 4, 8, 16, 32, 64):
            rowcum = rowcum + _lane_roll(rowcum, k, 0)
        rank_incl = cum + (rowcum - rows)
        masks.append((keys > v_thr) | ((ties > 0) & (rank_incl <= r)))
    return masks


def _select_kernel(sem_ref, p_ref, out_ref):
    sem0 = [[sem_ref[n, c] for c in range(_C)] for n in range(_N)]
    p = [[p_ref[n, c] for c in range(_C)] for n in range(_N)]
    k1 = [_keys_of(sem0[n]) for n in range(_N)]
    m1 = _topk_masks(k1)
    sem1 = [[jnp.where(m1[n], p[n][c], sem0[n][c]) for c in range(_C)]
            for n in range(_N)]
    k2 = [_keys_of(sem1[n]) for n in range(_N)]
    m2 = _topk_masks(k2)
    for n in range(_N):
        for c in range(_C):
            out_ref[n, c] = jnp.where(m2[n], p[n][c], sem1[n][c])


def kernel(pred_logits, features, fc1_w, fc1_b, fc2_w, fc2_b,
           fc3_w, fc3_b, pred_w, pred_b):
    bf = jnp.bfloat16
    wa1 = jnp.concatenate([fc1_w, fc1_b[:, None]], axis=1).astype(bf)
    wa2 = jnp.concatenate([fc2_w, fc2_b[:, None]], axis=1).astype(bf)
    wa3 = jnp.concatenate([fc3_w, fc3_b[:, None]], axis=1).astype(bf)
    wap = jnp.concatenate([pred_w, pred_b[:, None]], axis=1).astype(bf)

    full = lambda shape: pl.BlockSpec(shape, lambda n, j: tuple(0 for _ in shape))
    p4 = pl.pallas_call(
        _mlp_kernel,
        grid=(_N, _H // _ROWS),
        in_specs=[
            pl.BlockSpec((1, _F, _ROWS, _W), lambda n, j: (n, 0, j, 0)),
            pl.BlockSpec((1, _C, _ROWS, _W), lambda n, j: (n, 0, j, 0)),
            full((_F, _F + _C + 1)), full((_F, _F + _C + 1)),
            full((_F, _F + _C + 1)), full((_C, _F + _C + 1)),
        ],
        out_specs=pl.BlockSpec((1, _C, _ROWS, _W), lambda n, j: (n, 0, j, 0)),
        out_shape=jax.ShapeDtypeStruct((_N, _C, _H, _W), jnp.float32),
    )(features, pred_logits, wa1, wa2, wa3, wap)

    return pl.pallas_call(
        _select_kernel,
        out_shape=jax.ShapeDtypeStruct((_N, _C, _H, _W), jnp.float32),
    )(pred_logits, p4)


# speedup vs baseline: 1.0571x; 1.0571x over previous
import jax
import jax.numpy as jnp
from jax.experimental import pallas as pl
from jax.experimental.pallas import tpu as pltpu

_N, _C, _F, _H, _W = 2, 19, 256, 128, 128
_HW = _H * _W
_K = 8192
_BLK = 4096


_ROWS = _BLK // _W


def _mlp_block(feat_ref, pred_ref, wa1, wa2, wa3, wap, p_scr, n, j):
    bf = jnp.bfloat16
    nch = 2
    cr = _ROWS // nch
    cb = _BLK // nch
    w1, w2, w3, wp = wa1[...], wa2[...], wa3[...], wap[...]
    base = j * _ROWS

    def dot(a, b):
        return jax.lax.dot(a, b, preferred_element_type=jnp.float32)

    def prep(s):
        feat = feat_ref[0][:, s * cr:(s + 1) * cr].astype(bf).reshape(_F, cb)
        coarse = (pred_ref[n, :, pl.ds(base + s * cr, cr), :]
                  .astype(bf).reshape(_C, cb))
        tail = jnp.concatenate([coarse, jnp.ones((1, cb), bf)], axis=0)
        return jnp.concatenate([feat, tail], axis=0), tail

    x0, t0 = prep(0)
    d0 = dot(w1, x0)
    x1, t1 = prep(1)
    x0 = jnp.concatenate([jnp.maximum(d0, 0.0).astype(bf), t0], axis=0)
    d1 = dot(w1, x1)
    d0 = dot(w2, x0)
    x1 = jnp.concatenate([jnp.maximum(d1, 0.0).astype(bf), t1], axis=0)
    d1 = dot(w2, x1)
    x0 = jnp.concatenate([jnp.maximum(d0, 0.0).astype(bf), t0], axis=0)
    d0 = dot(w3, x0)
    x1 = jnp.concatenate([jnp.maximum(d1, 0.0).astype(bf), t1], axis=0)
    d1 = dot(w3, x1)
    x0 = jnp.concatenate([jnp.maximum(d0, 0.0).astype(bf), t0], axis=0)
    o0 = dot(wp, x0)
    x1 = jnp.concatenate([jnp.maximum(d1, 0.0).astype(bf), t1], axis=0)
    o1 = dot(wp, x1)
    p_scr[n, :, pl.ds(base, cr), :] = o0.reshape(_C, cr, _W)
    p_scr[n, :, pl.ds(base + cr, cr), :] = o1.reshape(_C, cr, _W)


def _keys_of(sem_list):
    minint = jnp.int32(-2147483648)
    m1 = sem_list[0]
    m2 = jnp.full((_H, _W), -jnp.inf, jnp.float32)
    for c in range(1, _C):
        v = sem_list[c]
        m2 = jnp.maximum(m2, jnp.minimum(m1, v))
        m1 = jnp.maximum(m1, v)
    unc = m2 - m1
    bits = jax.lax.bitcast_convert_type(unc, jnp.int32)
    return jnp.where(bits >= 0, bits, minint - bits)


def _lane_roll(x, k, axis):
    idx = jax.lax.broadcasted_iota(jnp.int32, (_H, _W), axis)
    return jnp.where(idx >= k, pltpu.roll(x, k, axis=axis), 0)


def _topk_masks(keys_pair):
    minint = jnp.int32(-2147483648)

    def t_body(i, ts):
        bit = jnp.left_shift(jnp.int32(1), jnp.int32(30) - i)
        new = []
        for keys, t in zip(keys_pair, ts):
            cand = t + bit
            cnt = jnp.sum((keys > cand).astype(jnp.int32))
            new.append(jnp.where(cnt >= _K, cand, t))
        return tuple(new)

    ts = jax.lax.fori_loop(0, 31, t_body, (minint, minint), unroll=True)

    masks = []
    for keys, t in zip(keys_pair, ts):
        v_thr = t + jnp.int32(1)
        g = jnp.sum((keys > v_thr).astype(jnp.int32))
        r = _K - g
        ties = (keys == v_thr).astype(jnp.int32)
        cum = ties
        for k in (1, 2, 4, 8, 16, 32, 64):
            cum = cum + _lane_roll(cum, k, 1)
        rows = jax.lax.broadcast_in_dim(cum[:, _W - 1], (_H, _W), (0,))
        rowcum = rows
        for k in (1, 2, 4, 8, 16, 32, 64):
            rowcum = rowcum + _lane_roll(rowcum, k, 0)
        rank_incl = cum + (rowcum - rows)
        masks.append((keys > v_thr) | ((ties > 0) & (rank_incl <= r)))
    return masks


def _select_phase(sem_ref, p_ref, out_ref):
    sem0 = [[sem_ref[n, c] for c in range(_C)] for n in range(_N)]
    p = [[p_ref[n, c] for c in range(_C)] for n in range(_N)]
    k1 = [_keys_of(sem0[n]) for n in range(_N)]
    m1 = _topk_masks(k1)
    sem1 = [[jnp.where(m1[n], p[n][c], sem0[n][c]) for c in range(_C)]
            for n in range(_N)]
    k2 = [_keys_of(sem1[n]) for n in range(_N)]
    m2 = _topk_masks(k2)
    for n in range(_N):
        for c in range(_C):
            out_ref[n, c] = jnp.where(m2[n], p[n][c], sem1[n][c])


_NBLK = _HW // _BLK


def _fused_kernel(feat_ref, pred_ref, wa1, wa2, wa3, wap, out_ref, p_scr):
    i = pl.program_id(0)

    @pl.when(i < _N * _NBLK)
    def _mlp():
        _mlp_block(feat_ref, pred_ref, wa1, wa2, wa3, wap, p_scr,
                   i // _NBLK, i % _NBLK)

    @pl.when(i == _N * _NBLK)
    def _sel():
        _select_phase(pred_ref, p_scr, out_ref)


def kernel(pred_logits, features, fc1_w, fc1_b, fc2_w, fc2_b,
           fc3_w, fc3_b, pred_w, pred_b):
    bf = jnp.bfloat16
    wa1 = jnp.concatenate([fc1_w, fc1_b[:, None]], axis=1).astype(bf)
    wa2 = jnp.concatenate([fc2_w, fc2_b[:, None]], axis=1).astype(bf)
    wa3 = jnp.concatenate([fc3_w, fc3_b[:, None]], axis=1).astype(bf)
    wap = jnp.concatenate([pred_w, pred_b[:, None]], axis=1).astype(bf)

    full = lambda shape: pl.BlockSpec(shape, lambda i: tuple(0 for _ in shape))
    nsteps = _N * _NBLK

    def feat_map(i):
        i = jnp.minimum(i, nsteps - 1)
        return (i // _NBLK, 0, i % _NBLK, 0)

    return pl.pallas_call(
        _fused_kernel,
        grid=(nsteps + 1,),
        in_specs=[
            pl.BlockSpec((1, _F, _ROWS, _W), feat_map),
            full((_N, _C, _H, _W)),
            full((_F, _F + _C + 1)), full((_F, _F + _C + 1)),
            full((_F, _F + _C + 1)), full((_C, _F + _C + 1)),
        ],
        out_specs=full((_N, _C, _H, _W)),
        out_shape=jax.ShapeDtypeStruct((_N, _C, _H, _W), jnp.float32),
        scratch_shapes=[pltpu.VMEM((_N, _C, _H, _W), jnp.float32)],
    )(features, pred_logits, wa1, wa2, wa3, wap)


# speedup vs baseline: 1.1103x; 1.0503x over previous
import jax
import jax.numpy as jnp
from jax.experimental import pallas as pl
from jax.experimental.pallas import tpu as pltpu

_N, _C, _F, _H, _W = 2, 19, 256, 128, 128
_HW = _H * _W
_K = 8192
_BLK = 4096


_ROWS = _BLK // _W


def _mlp_block(feat_ref, pred_ref, wa1, wa2, wa3, wap, p_scr, n, j):
    bf = jnp.bfloat16
    nch = 2
    cr = _ROWS // nch
    cb = _BLK // nch
    w1, w2, w3, wp = wa1[...], wa2[...], wa3[...], wap[...]
    base = j * _ROWS

    def dot(a, b):
        return jax.lax.dot(a, b, preferred_element_type=jnp.float32)

    def prep(s):
        feat = feat_ref[0][:, s * cr:(s + 1) * cr].astype(bf).reshape(_F, cb)
        coarse = (pred_ref[n, :, pl.ds(base + s * cr, cr), :]
                  .astype(bf).reshape(_C, cb))
        tail = jnp.concatenate([coarse, jnp.ones((1, cb), bf)], axis=0)
        return jnp.concatenate([feat, tail], axis=0), tail

    x0, t0 = prep(0)
    d0 = dot(w1, x0)
    x1, t1 = prep(1)
    x0 = jnp.concatenate([jnp.maximum(d0, 0.0).astype(bf), t0], axis=0)
    d1 = dot(w1, x1)
    d0 = dot(w2, x0)
    x1 = jnp.concatenate([jnp.maximum(d1, 0.0).astype(bf), t1], axis=0)
    d1 = dot(w2, x1)
    x0 = jnp.concatenate([jnp.maximum(d0, 0.0).astype(bf), t0], axis=0)
    d0 = dot(w3, x0)
    x1 = jnp.concatenate([jnp.maximum(d1, 0.0).astype(bf), t1], axis=0)
    d1 = dot(w3, x1)
    x0 = jnp.concatenate([jnp.maximum(d0, 0.0).astype(bf), t0], axis=0)
    o0 = dot(wp, x0)
    x1 = jnp.concatenate([jnp.maximum(d1, 0.0).astype(bf), t1], axis=0)
    o1 = dot(wp, x1)
    p_scr[n, :, pl.ds(base, cr), :] = o0.reshape(_C, cr, _W)
    p_scr[n, :, pl.ds(base + cr, cr), :] = o1.reshape(_C, cr, _W)


def _keys_of(sem_list):
    minint = jnp.int32(-2147483648)
    m1 = sem_list[0]
    m2 = jnp.full((_H, _W), -jnp.inf, jnp.float32)
    for c in range(1, _C):
        v = sem_list[c]
        m2 = jnp.maximum(m2, jnp.minimum(m1, v))
        m1 = jnp.maximum(m1, v)
    unc = m2 - m1
    bits = jax.lax.bitcast_convert_type(unc, jnp.int32)
    return jnp.where(bits >= 0, bits, minint - bits)


def _lane_roll(x, k, axis):
    idx = jax.lax.broadcasted_iota(jnp.int32, (_H, _W), axis)
    return jnp.where(idx >= k, pltpu.roll(x, k, axis=axis), 0)


def _topk_masks(keys_pair):
    minint = jnp.int32(-2147483648)

    def t_body(i, ts):
        step = jnp.left_shift(jnp.int32(1), jnp.int32(30) - 2 * i)
        new = []
        for keys, t in zip(keys_pair, ts):
            c1 = t + step
            c2 = c1 + step
            c3 = c2 + step
            n1 = jnp.sum((keys > c1).astype(jnp.int32))
            n2 = jnp.sum((keys > c2).astype(jnp.int32))
            n3 = jnp.sum((keys > c3).astype(jnp.int32))
            adv = ((n1 >= _K).astype(jnp.int32)
                   + (n2 >= _K).astype(jnp.int32)
                   + (n3 >= _K).astype(jnp.int32))
            new.append(t + step * adv)
        return tuple(new)

    ts = jax.lax.fori_loop(0, 16, t_body, (minint, minint), unroll=True)

    masks = []
    for keys, t in zip(keys_pair, ts):
        v_thr = t + jnp.int32(1)
        g = jnp.sum((keys > v_thr).astype(jnp.int32))
        r = _K - g
        ties = (keys == v_thr).astype(jnp.int32)
        cum = ties
        for k in (1, 2, 4, 8, 16, 32, 64):
            cum = cum + _lane_roll(cum, k, 1)
        rows = jax.lax.broadcast_in_dim(cum[:, _W - 1], (_H, _W), (0,))
        rowcum = rows
        for k in (1, 2, 4, 8, 16, 32, 64):
            rowcum = rowcum + _lane_roll(rowcum, k, 0)
        rank_incl = cum + (rowcum - rows)
        masks.append((keys > v_thr) | ((ties > 0) & (rank_incl <= r)))
    return masks


def _select_phase(sem_ref, p_ref, out_ref):
    sem0 = [[sem_ref[n, c] for c in range(_C)] for n in range(_N)]
    p = [[p_ref[n, c] for c in range(_C)] for n in range(_N)]
    k1 = [_keys_of(sem0[n]) for n in range(_N)]
    m1 = _topk_masks(k1)
    sem1 = [[jnp.where(m1[n], p[n][c], sem0[n][c]) for c in range(_C)]
            for n in range(_N)]
    k2 = [_keys_of(sem1[n]) for n in range(_N)]
    m2 = _topk_masks(k2)
    for n in range(_N):
        for c in range(_C):
            out_ref[n, c] = jnp.where(m2[n], p[n][c], sem1[n][c])


_NBLK = _HW // _BLK


def _fused_kernel(feat_ref, pred_ref, wa1, wa2, wa3, wap, out_ref, p_scr):
    i = pl.program_id(0)

    @pl.when(i < _N * _NBLK)
    def _mlp():
        _mlp_block(feat_ref, pred_ref, wa1, wa2, wa3, wap, p_scr,
                   i // _NBLK, i % _NBLK)

    @pl.when(i == _N * _NBLK)
    def _sel():
        _select_phase(pred_ref, p_scr, out_ref)


def kernel(pred_logits, features, fc1_w, fc1_b, fc2_w, fc2_b,
           fc3_w, fc3_b, pred_w, pred_b):
    bf = jnp.bfloat16
    wa1 = jnp.concatenate([fc1_w, fc1_b[:, None]], axis=1).astype(bf)
    wa2 = jnp.concatenate([fc2_w, fc2_b[:, None]], axis=1).astype(bf)
    wa3 = jnp.concatenate([fc3_w, fc3_b[:, None]], axis=1).astype(bf)
    wap = jnp.concatenate([pred_w, pred_b[:, None]], axis=1).astype(bf)

    full = lambda shape: pl.BlockSpec(shape, lambda i: tuple(0 for _ in shape))
    nsteps = _N * _NBLK

    def feat_map(i):
        i = jnp.minimum(i, nsteps - 1)
        return (i // _NBLK, 0, i % _NBLK, 0)

    return pl.pallas_call(
        _fused_kernel,
        grid=(nsteps + 1,),
        in_specs=[
            pl.BlockSpec((1, _F, _ROWS, _W), feat_map),
            full((_N, _C, _H, _W)),
            full((_F, _F + _C + 1)), full((_F, _F + _C + 1)),
            full((_F, _F + _C + 1)), full((_C, _F + _C + 1)),
        ],
        out_specs=full((_N, _C, _H, _W)),
        out_shape=jax.ShapeDtypeStruct((_N, _C, _H, _W), jnp.float32),
        scratch_shapes=[pltpu.VMEM((_N, _C, _H, _W), jnp.float32)],
    )(features, pred_logits, wa1, wa2, wa3, wap)
